# baseline (speedup 1.0000x reference)
import functools

import jax
import jax.numpy as jnp
from jax import lax
from jax.experimental import pallas as pl
from jax.experimental.pallas import tpu as pltpu
from jax.experimental.pallas import tpu_sc as plsc

B = 16384
D = 64
NNEG = 20
NV = NNEG + 1
L = 16
NC = 2
NS = 16
NW = NC * NS
BW = B // NW
CB = 32
NCHUNK = BW // CB
GJ = 6
GN = CB * NV // GJ
W128 = 2 * D


def _sc_body(upos_hbm, vidx_hbm, uw_hbm, vw_hbm, pos_hbm, neg_hbm,
             uidx_v, urow_v, vidx_v, vrow_v, urows, vrows, posb, negb, sem):
    wid = lax.axis_index("s") * NC + lax.axis_index("c")

    def chunk_body(c, carry):
        gbase = wid * BW + c * CB

        pltpu.sync_copy(upos_hbm.at[pl.ds(gbase, CB)], uidx_v.at[pl.ds(0, CB)])
        pltpu.sync_copy(vidx_hbm.at[pl.ds(gbase * NV, CB * NV)],
                        vidx_v.at[pl.ds(0, CB * NV)])

        def to_row(x):
            return lax.shift_left(lax.shift_right_logical(x, 11), 10) | (
                x & 1023)

        for i in range(CB // L):
            x = uidx_v[pl.ds(i * L, L)]
            urow_v[pl.ds(i * L, L)] = to_row(x)
        for i in range(CB * NV // L):
            x = vidx_v[pl.ds(i * L, L)]
            vrow_v[pl.ds(i * L, L)] = to_row(x)

        copies = [pltpu.async_copy(uw_hbm.at[urow_v], urows, sem)]
        for j in range(GJ):
            copies.append(pltpu.async_copy(
                vw_hbm.at[vrow_v.at[pl.ds(j * GN, GN)]],
                vrows.at[pl.ds(j * GN, GN)], sem))
        for cp in copies:
            cp.wait()

        def bbody(b, carry2):
            rb = b * NV

            def half_off(pe):
                return (lax.shift_right_logical(pe, 10) & 1) * D

            upar = uidx_v[pl.ds(b, L)]
            uoff = half_off(upar[0])
            u = [urows[b, pl.ds(uoff + 16 * k, 16)] for k in range(4)]
            vpar = vidx_v[pl.ds(rb, L)]
            voff = half_off(vpar[0])
            v = [vrows[rb, pl.ds(voff + 16 * k, 16)] for k in range(4)]
            aoff = half_off(vpar[1])
            acc = [vrows[rb + 1, pl.ds(aoff + 16 * k, 16)] for k in range(4)]
            vpar2 = vidx_v[pl.ds(rb + L, L)]
            for n in range(2, NV):
                pe = vpar[n] if n < L else vpar2[n - L]
                noff = half_off(pe)
                for k in range(4):
                    acc[k] = acc[k] + vrows[rb + n, pl.ds(noff + 16 * k, 16)]
            pos = u[0] * v[0] + u[1] * v[1] + u[2] * v[2] + u[3] * v[3]
            neg = u[0] * acc[0] + u[1] * acc[1] + u[2] * acc[2] + u[3] * acc[3]
            posb[pl.ds(b * L, L)] = pos
            negb[pl.ds(b * L, L)] = neg
            return carry2

        lax.fori_loop(0, CB, bbody, 0, unroll=False)

        pltpu.sync_copy(posb, pos_hbm.at[pl.ds(gbase * L, CB * L)])
        pltpu.sync_copy(negb, neg_hbm.at[pl.ds(gbase * L, CB * L)])
        return carry

    lax.fori_loop(0, NCHUNK, chunk_body, 0, unroll=False)


_sc_call = functools.partial(
    pl.kernel,
    out_type=(jax.ShapeDtypeStruct((B * L,), jnp.float32),
              jax.ShapeDtypeStruct((B * L,), jnp.float32)),
    mesh=plsc.VectorSubcoreMesh(core_axis_name="c", subcore_axis_name="s"),
    compiler_params=pltpu.CompilerParams(use_tc_tiling_on_sc=True),
    scratch_types=[
        pltpu.VMEM((CB + L,), jnp.int32),
        pltpu.VMEM((CB,), jnp.int32),
        pltpu.VMEM((CB * NV + 2 * L,), jnp.int32),
        pltpu.VMEM((CB * NV,), jnp.int32),
        pltpu.VMEM((CB, W128), jnp.float32),
        pltpu.VMEM((CB * NV, W128), jnp.float32),
        pltpu.VMEM((CB * L,), jnp.float32),
        pltpu.VMEM((CB * L,), jnp.float32),
        pltpu.SemaphoreType.DMA,
    ],
)(_sc_body)


VB = 1024
NGB = 489
TROWS = NGB * VB


def _tr_body(ua_ref, ub_ref, uo_ref):
    ii = lax.broadcasted_iota(jnp.int32, (D, W128), 0)
    jj = lax.broadcasted_iota(jnp.int32, (D, W128), 1)
    sel_lo = (ii == jj).astype(jnp.float32)
    sel_hi = (ii == jj - D).astype(jnp.float32)
    dims = (((0,), (0,)), ((), ()))
    uo_ref[...] = (
        lax.dot_general(ua_ref[...], sel_lo, dims,
                        preferred_element_type=jnp.float32)
        + lax.dot_general(ub_ref[...], sel_hi, dims,
                          preferred_element_type=jnp.float32))


_tr_call = pl.pallas_call(
    _tr_body,
    grid=(NGB,),
    in_specs=[pl.BlockSpec((D, VB), lambda j: (0, 2 * j)),
              pl.BlockSpec((D, VB), lambda j: (0, jnp.minimum(2 * j + 1, 976)))],
    out_specs=pl.BlockSpec((VB, W128), lambda j: (j, 0)),
    out_shape=jax.ShapeDtypeStruct((TROWS, W128), jnp.float32),
)


NTILE = TROWS // 128
TPW = 124


def _vrep_body(vt_hbm, vo_hbm, loA, hiA, outA, loB, hiB, outB, semA, semB):
    wid = lax.axis_index("s") * NC + lax.axis_index("c")
    riv = [lax.iota(jnp.int32, L) + 16 * k for k in range(4)]

    def cols_of(k):
        t = jnp.minimum(wid + 32 * k, NTILE - 2)
        col0 = 128 * t + 1024 * lax.shift_right_logical(t, 3)
        col_lo = jnp.minimum(col0, 999872)
        col_hi = jnp.minimum(col0 + 1024, 998912)
        return t, col0, col_lo, col_hi

    def fire_in(k, lob, hib, sem):
        _, _, col_lo, col_hi = cols_of(k)
        pltpu.async_copy(
            vt_hbm.at[:, pl.ds(pl.multiple_of(col_lo, 128), 128)], lob, sem)
        pltpu.async_copy(
            vt_hbm.at[:, pl.ds(pl.multiple_of(col_hi, 128), 128)], hib, sem)

    def wait_in(lob, hib, sem):
        pltpu.make_async_copy(vt_hbm.at[:, pl.ds(0, 128)], lob, sem).wait()
        pltpu.make_async_copy(vt_hbm.at[:, pl.ds(0, 128)], hib, sem).wait()

    def compute(k, lob, hib, outb):
        t, col0, col_lo, _ = cols_of(k)
        shift = jnp.minimum(col0 - col_lo, 64)

        def row(i, carry):
            ci = jnp.full((L,), jnp.minimum(i + shift, 127), jnp.int32)
            ch = jnp.full((L,), i, jnp.int32)
            for kk in range(4):
                outb[i, pl.ds(16 * kk, 16)] = plsc.load_gather(
                    lob, [riv[kk], ci])
                outb[i, pl.ds(D + 16 * kk, 16)] = plsc.load_gather(
                    hib, [riv[kk], ch])
            return carry

        lax.fori_loop(0, 128, row, 0, unroll=False)
        pltpu.sync_copy(
            outb, vo_hbm.at[pl.ds(pl.multiple_of(t * 128, 128), 128)])

    fire_in(0, loA, hiA, semA)

    def pair(k2, carry):
        fire_in(2 * k2 + 1, loB, hiB, semB)
        wait_in(loA, hiA, semA)
        compute(2 * k2, loA, hiA, outA)
        fire_in(2 * k2 + 2, loA, hiA, semA)
        wait_in(loB, hiB, semB)
        compute(2 * k2 + 1, loB, hiB, outB)
        return carry

    lax.fori_loop(0, TPW // 2, pair, 0, unroll=False)
    wait_in(loA, hiA, semA)


_vrep_call = functools.partial(
    pl.kernel,
    out_type=jax.ShapeDtypeStruct((TROWS, W128), jnp.float32),
    mesh=plsc.VectorSubcoreMesh(core_axis_name="c", subcore_axis_name="s"),
    compiler_params=pltpu.CompilerParams(use_tc_tiling_on_sc=True,
                                         needs_layout_passes=False),
    scratch_types=[
        pltpu.VMEM((D, 128), jnp.float32),
        pltpu.VMEM((D, 128), jnp.float32),
        pltpu.VMEM((128, W128), jnp.float32),
        pltpu.VMEM((D, 128), jnp.float32),
        pltpu.VMEM((D, 128), jnp.float32),
        pltpu.VMEM((128, W128), jnp.float32),
        pltpu.SemaphoreType.DMA,
        pltpu.SemaphoreType.DMA,
    ],
)(_vrep_body)


def _loss_body(pos_ref, neg_ref, out_ref):
    score = jnp.sum(pos_ref[...], axis=1)
    nscore = jnp.sum(neg_ref[...], axis=1)

    def logsig(x):
        return jnp.minimum(x, 0.0) - jnp.log1p(jnp.exp(-jnp.abs(x)))

    out_ref[0, 0] = -jnp.mean(logsig(score) + logsig(-nscore))


_loss_call = pl.pallas_call(
    _loss_body,
    out_shape=jax.ShapeDtypeStruct((1, 1), jnp.float32),
    out_specs=pl.BlockSpec(memory_space=pltpu.SMEM),
)


def kernel(u_pos, v_pos, v_neg, u_weight, v_weight):
    vidx = jnp.concatenate([v_pos[:, None], v_neg], axis=1).reshape(-1)
    uwT, vwT = u_weight.T, v_weight.T
    vw2 = _vrep_call(vwT)
    uw2 = _tr_call(uwT, uwT)
    pos_flat, neg_flat = _sc_call(u_pos, vidx, uw2, vw2)
    out = _loss_call(pos_flat.reshape(B, L), neg_flat.reshape(B, L))
    return out[0, 0]


# speedup vs baseline: 2.7528x; 2.7528x over previous
import functools

import jax
import jax.numpy as jnp
from jax import lax
from jax.experimental import pallas as pl
from jax.experimental.pallas import tpu as pltpu
from jax.experimental.pallas import tpu_sc as plsc

B = 16384
D = 64
NNEG = 20
NV = NNEG + 1
L = 16
NC = 2
NS = 16
NW = NC * NS
BW = B // NW
CB = 32
NCHUNK = BW // CB
GJ = 6
GN = CB * NV // GJ
W128 = 2 * D


def _sc_body(upos_hbm, vidx_hbm, uw_hbm, vw_hbm, pos_hbm, neg_hbm,
             uidx_v, urow_v, vidx_v, vrow_v, urows, vrows, posb, negb, sem):
    wid = lax.axis_index("s") * NC + lax.axis_index("c")

    def chunk_body(c, carry):
        gbase = wid * BW + c * CB

        pltpu.sync_copy(upos_hbm.at[pl.ds(gbase, CB)], uidx_v.at[pl.ds(0, CB)])
        pltpu.sync_copy(vidx_hbm.at[pl.ds(gbase * NV, CB * NV)],
                        vidx_v.at[pl.ds(0, CB * NV)])

        def to_row(x):
            return lax.shift_left(lax.shift_right_logical(x, 12), 11) | (
                x & 2047)

        for i in range(CB // L):
            x = uidx_v[pl.ds(i * L, L)]
            urow_v[pl.ds(i * L, L)] = to_row(x)
        for i in range(CB * NV // L):
            x = vidx_v[pl.ds(i * L, L)]
            vrow_v[pl.ds(i * L, L)] = to_row(x)

        copies = [pltpu.async_copy(uw_hbm.at[urow_v], urows, sem)]
        for j in range(GJ):
            copies.append(pltpu.async_copy(
                vw_hbm.at[vrow_v.at[pl.ds(j * GN, GN)]],
                vrows.at[pl.ds(j * GN, GN)], sem))
        for cp in copies:
            cp.wait()

        def bbody(b, carry2):
            rb = b * NV

            def half_off(pe):
                return (lax.shift_right_logical(pe, 11) & 1) * D

            upar = uidx_v[pl.ds(b, L)]
            uoff = half_off(upar[0])
            u = [urows[b, pl.ds(uoff + 16 * k, 16)] for k in range(4)]
            vpar = vidx_v[pl.ds(rb, L)]
            voff = half_off(vpar[0])
            v = [vrows[rb, pl.ds(voff + 16 * k, 16)] for k in range(4)]
            aoff = half_off(vpar[1])
            acc = [vrows[rb + 1, pl.ds(aoff + 16 * k, 16)] for k in range(4)]
            vpar2 = vidx_v[pl.ds(rb + L, L)]
            for n in range(2, NV):
                pe = vpar[n] if n < L else vpar2[n - L]
                noff = half_off(pe)
                for k in range(4):
                    acc[k] = acc[k] + vrows[rb + n, pl.ds(noff + 16 * k, 16)]
            pos = u[0] * v[0] + u[1] * v[1] + u[2] * v[2] + u[3] * v[3]
            neg = u[0] * acc[0] + u[1] * acc[1] + u[2] * acc[2] + u[3] * acc[3]
            posb[pl.ds(b * L, L)] = pos
            negb[pl.ds(b * L, L)] = neg
            return carry2

        lax.fori_loop(0, CB, bbody, 0, unroll=False)

        pltpu.sync_copy(posb, pos_hbm.at[pl.ds(gbase * L, CB * L)])
        pltpu.sync_copy(negb, neg_hbm.at[pl.ds(gbase * L, CB * L)])
        return carry

    lax.fori_loop(0, NCHUNK, chunk_body, 0, unroll=False)


_sc_call = functools.partial(
    pl.kernel,
    out_type=(jax.ShapeDtypeStruct((B * L,), jnp.float32),
              jax.ShapeDtypeStruct((B * L,), jnp.float32)),
    mesh=plsc.VectorSubcoreMesh(core_axis_name="c", subcore_axis_name="s"),
    compiler_params=pltpu.CompilerParams(use_tc_tiling_on_sc=True),
    scratch_types=[
        pltpu.VMEM((CB + L,), jnp.int32),
        pltpu.VMEM((CB,), jnp.int32),
        pltpu.VMEM((CB * NV + 2 * L,), jnp.int32),
        pltpu.VMEM((CB * NV,), jnp.int32),
        pltpu.VMEM((CB, W128), jnp.float32),
        pltpu.VMEM((CB * NV, W128), jnp.float32),
        pltpu.VMEM((CB * L,), jnp.float32),
        pltpu.VMEM((CB * L,), jnp.float32),
        pltpu.SemaphoreType.DMA,
    ],
)(_sc_body)


VB = 2048
NGB = 245
TROWS = NGB * VB


def _tr_body(ua_ref, ub_ref, va_ref, vb_ref, uo_ref, vo_ref):
    ii = lax.broadcasted_iota(jnp.int32, (D, W128), 0)
    jj = lax.broadcasted_iota(jnp.int32, (D, W128), 1)
    sel_lo = (ii == jj).astype(jnp.float32)
    sel_hi = (ii == jj - D).astype(jnp.float32)
    dims = (((0,), (0,)), ((), ()))
    for lo, hi, dst in ((ua_ref, ub_ref, uo_ref), (va_ref, vb_ref, vo_ref)):
        dst[...] = (
            lax.dot_general(lo[...], sel_lo, dims,
                            preferred_element_type=jnp.float32)
            + lax.dot_general(hi[...], sel_hi, dims,
                              preferred_element_type=jnp.float32))


_tr_call = pl.pallas_call(
    _tr_body,
    grid=(NGB,),
    in_specs=[pl.BlockSpec((D, VB), lambda j: (0, 2 * j)),
              pl.BlockSpec((D, VB), lambda j: (0, jnp.minimum(2 * j + 1, 488))),
              pl.BlockSpec((D, VB), lambda j: (0, 2 * j)),
              pl.BlockSpec((D, VB), lambda j: (0, jnp.minimum(2 * j + 1, 488)))],
    out_specs=[pl.BlockSpec((VB, W128), lambda j: (j, 0)),
               pl.BlockSpec((VB, W128), lambda j: (j, 0))],
    out_shape=[jax.ShapeDtypeStruct((TROWS, W128), jnp.float32),
               jax.ShapeDtypeStruct((TROWS, W128), jnp.float32)],
)


def _loss_body(pos_ref, neg_ref, out_ref):
    score = jnp.sum(pos_ref[...], axis=1)
    nscore = jnp.sum(neg_ref[...], axis=1)

    def logsig(x):
        return jnp.minimum(x, 0.0) - jnp.log1p(jnp.exp(-jnp.abs(x)))

    out_ref[0, 0] = -jnp.mean(logsig(score) + logsig(-nscore))


_loss_call = pl.pallas_call(
    _loss_body,
    out_shape=jax.ShapeDtypeStruct((1, 1), jnp.float32),
    out_specs=pl.BlockSpec(memory_space=pltpu.SMEM),
)


def kernel(u_pos, v_pos, v_neg, u_weight, v_weight):
    vidx = jnp.concatenate([v_pos[:, None], v_neg], axis=1).reshape(-1)
    uwT, vwT = u_weight.T, v_weight.T
    uw2, vw2 = _tr_call(uwT, uwT, vwT, vwT)
    pos_flat, neg_flat = _sc_call(u_pos, vidx, uw2, vw2)
    out = _loss_call(pos_flat.reshape(B, L), neg_flat.reshape(B, L))
    return out[0, 0]


# speedup vs baseline: 3.1111x; 1.1302x over previous
import functools

import jax
import jax.numpy as jnp
from jax import lax
from jax.experimental import pallas as pl
from jax.experimental.pallas import tpu as pltpu
from jax.experimental.pallas import tpu_sc as plsc

B = 16384
D = 64
NNEG = 20
NV = NNEG + 1
L = 16
NC = 2
NS = 16
NW = NC * NS
BW = B // NW
CB = 32
NCHUNK = BW // CB
GJ = 6
GN = CB * NV // GJ
W128 = 2 * D


def _sc_body(upos_hbm, vidx_hbm, uw_hbm, vw_hbm, pos_hbm, neg_hbm,
             uidx_v, urow_v, vidx_v, vrow_v, urows, vrows, posb, negb, sem):
    wid = lax.axis_index("s") * NC + lax.axis_index("c")

    def chunk_body(c, carry):
        gbase = wid * BW + c * CB

        pltpu.sync_copy(upos_hbm.at[pl.ds(gbase, CB)], uidx_v.at[pl.ds(0, CB)])
        pltpu.sync_copy(vidx_hbm.at[pl.ds(gbase * NV, CB * NV)],
                        vidx_v.at[pl.ds(0, CB * NV)])

        def to_row(x):
            return lax.shift_left(lax.shift_right_logical(x, 13), 12) | (
                x & 4095)

        for i in range(CB // L):
            x = uidx_v[pl.ds(i * L, L)]
            urow_v[pl.ds(i * L, L)] = to_row(x)
        for i in range(CB * NV // L):
            x = vidx_v[pl.ds(i * L, L)]
            vrow_v[pl.ds(i * L, L)] = to_row(x)

        copies = [pltpu.async_copy(uw_hbm.at[urow_v], urows, sem)]
        for j in range(GJ):
            copies.append(pltpu.async_copy(
                vw_hbm.at[vrow_v.at[pl.ds(j * GN, GN)]],
                vrows.at[pl.ds(j * GN, GN)], sem))
        for cp in copies:
            cp.wait()

        def bbody(b, carry2):
            rb = b * NV

            def half_off(pe):
                return (lax.shift_right_logical(pe, 12) & 1) * D

            upar = uidx_v[pl.ds(b, L)]
            uoff = half_off(upar[0])
            u = [urows[b, pl.ds(uoff + 16 * k, 16)] for k in range(4)]
            vpar = vidx_v[pl.ds(rb, L)]
            voff = half_off(vpar[0])
            v = [vrows[rb, pl.ds(voff + 16 * k, 16)] for k in range(4)]
            aoff = half_off(vpar[1])
            acc = [vrows[rb + 1, pl.ds(aoff + 16 * k, 16)] for k in range(4)]
            vpar2 = vidx_v[pl.ds(rb + L, L)]
            for n in range(2, NV):
                pe = vpar[n] if n < L else vpar2[n - L]
                noff = half_off(pe)
                for k in range(4):
                    acc[k] = acc[k] + vrows[rb + n, pl.ds(noff + 16 * k, 16)]
            pos = u[0] * v[0] + u[1] * v[1] + u[2] * v[2] + u[3] * v[3]
            neg = u[0] * acc[0] + u[1] * acc[1] + u[2] * acc[2] + u[3] * acc[3]
            posb[pl.ds(b * L, L)] = pos
            negb[pl.ds(b * L, L)] = neg
            return carry2

        lax.fori_loop(0, CB, bbody, 0, unroll=False)

        pltpu.sync_copy(posb, pos_hbm.at[pl.ds(gbase * L, CB * L)])
        pltpu.sync_copy(negb, neg_hbm.at[pl.ds(gbase * L, CB * L)])
        return carry

    lax.fori_loop(0, NCHUNK, chunk_body, 0, unroll=False)


_sc_call = functools.partial(
    pl.kernel,
    out_type=(jax.ShapeDtypeStruct((B * L,), jnp.float32),
              jax.ShapeDtypeStruct((B * L,), jnp.float32)),
    mesh=plsc.VectorSubcoreMesh(core_axis_name="c", subcore_axis_name="s"),
    compiler_params=pltpu.CompilerParams(use_tc_tiling_on_sc=True),
    scratch_types=[
        pltpu.VMEM((CB + L,), jnp.int32),
        pltpu.VMEM((CB,), jnp.int32),
        pltpu.VMEM((CB * NV + 2 * L,), jnp.int32),
        pltpu.VMEM((CB * NV,), jnp.int32),
        pltpu.VMEM((CB, W128), jnp.float32),
        pltpu.VMEM((CB * NV, W128), jnp.float32),
        pltpu.VMEM((CB * L,), jnp.float32),
        pltpu.VMEM((CB * L,), jnp.float32),
        pltpu.SemaphoreType.DMA,
    ],
)(_sc_body)


VB = 4096
NGB = 123
TROWS = NGB * VB


def _tr_body(ua_ref, ub_ref, va_ref, vb_ref, uo_ref, vo_ref):
    ii = lax.broadcasted_iota(jnp.int32, (D, W128), 0)
    jj = lax.broadcasted_iota(jnp.int32, (D, W128), 1)
    sel_lo = (ii == jj).astype(jnp.float32)
    sel_hi = (ii == jj - D).astype(jnp.float32)
    dims = (((0,), (0,)), ((), ()))
    for lo, hi, dst in ((ua_ref, ub_ref, uo_ref), (va_ref, vb_ref, vo_ref)):
        dst[...] = (
            lax.dot_general(lo[...], sel_lo, dims,
                            preferred_element_type=jnp.float32)
            + lax.dot_general(hi[...], sel_hi, dims,
                              preferred_element_type=jnp.float32))


_tr_call = pl.pallas_call(
    _tr_body,
    grid=(NGB,),
    in_specs=[pl.BlockSpec((D, VB), lambda j: (0, 2 * j)),
              pl.BlockSpec((D, VB), lambda j: (0, jnp.minimum(2 * j + 1, 244))),
              pl.BlockSpec((D, VB), lambda j: (0, 2 * j)),
              pl.BlockSpec((D, VB), lambda j: (0, jnp.minimum(2 * j + 1, 244)))],
    out_specs=[pl.BlockSpec((VB, W128), lambda j: (j, 0)),
               pl.BlockSpec((VB, W128), lambda j: (j, 0))],
    out_shape=[jax.ShapeDtypeStruct((TROWS, W128), jnp.float32),
               jax.ShapeDtypeStruct((TROWS, W128), jnp.float32)],
)


def _loss_body(pos_ref, neg_ref, out_ref):
    score = jnp.sum(pos_ref[...], axis=1)
    nscore = jnp.sum(neg_ref[...], axis=1)

    def logsig(x):
        return jnp.minimum(x, 0.0) - jnp.log1p(jnp.exp(-jnp.abs(x)))

    out_ref[0, 0] = -jnp.mean(logsig(score) + logsig(-nscore))


_loss_call = pl.pallas_call(
    _loss_body,
    out_shape=jax.ShapeDtypeStruct((1, 1), jnp.float32),
    out_specs=pl.BlockSpec(memory_space=pltpu.SMEM),
)


def kernel(u_pos, v_pos, v_neg, u_weight, v_weight):
    vidx = jnp.concatenate([v_pos[:, None], v_neg], axis=1).reshape(-1)
    uwT, vwT = u_weight.T, v_weight.T
    uw2, vw2 = _tr_call(uwT, uwT, vwT, vwT)
    pos_flat, neg_flat = _sc_call(u_pos, vidx, uw2, vw2)
    out = _loss_call(pos_flat.reshape(B, L), neg_flat.reshape(B, L))
    return out[0, 0]


# speedup vs baseline: 3.3489x; 1.0764x over previous
import functools

import jax
import jax.numpy as jnp
from jax import lax
from jax.experimental import pallas as pl
from jax.experimental.pallas import tpu as pltpu
from jax.experimental.pallas import tpu_sc as plsc

B = 16384
D = 64
NNEG = 20
NV = NNEG + 1
L = 16
NC = 2
NS = 16
NW = NC * NS
BW = B // NW
CB = 32
NCHUNK = BW // CB
GJ = 6
GN = CB * NV // GJ
W128 = 2 * D


def _sc_body(upos_hbm, vidx_hbm, uw_hbm, vw_hbm, pos_hbm, neg_hbm,
             uidx_v, urow_v, vidx_v, vrow_v, urows, vrows, posb, negb, sem):
    wid = lax.axis_index("s") * NC + lax.axis_index("c")

    def chunk_body(c, carry):
        gbase = wid * BW + c * CB

        pltpu.sync_copy(upos_hbm.at[pl.ds(gbase, CB)], uidx_v.at[pl.ds(0, CB)])
        pltpu.sync_copy(vidx_hbm.at[pl.ds(gbase * NV, CB * NV)],
                        vidx_v.at[pl.ds(0, CB * NV)])

        def to_row(x):
            return lax.shift_left(lax.shift_right_logical(x, 14), 13) | (
                x & 8191)

        for i in range(CB // L):
            x = uidx_v[pl.ds(i * L, L)]
            urow_v[pl.ds(i * L, L)] = to_row(x)
        for i in range(CB * NV // L):
            x = vidx_v[pl.ds(i * L, L)]
            vrow_v[pl.ds(i * L, L)] = to_row(x)

        copies = [pltpu.async_copy(uw_hbm.at[urow_v], urows, sem)]
        for j in range(GJ):
            copies.append(pltpu.async_copy(
                vw_hbm.at[vrow_v.at[pl.ds(j * GN, GN)]],
                vrows.at[pl.ds(j * GN, GN)], sem))
        for cp in copies:
            cp.wait()

        def bbody(b, carry2):
            rb = b * NV

            def half_off(pe):
                return (lax.shift_right_logical(pe, 13) & 1) * D

            upar = uidx_v[pl.ds(b, L)]
            uoff = half_off(upar[0])
            u = [urows[b, pl.ds(uoff + 16 * k, 16)] for k in range(4)]
            vpar = vidx_v[pl.ds(rb, L)]
            voff = half_off(vpar[0])
            v = [vrows[rb, pl.ds(voff + 16 * k, 16)] for k in range(4)]
            aoff = half_off(vpar[1])
            acc = [vrows[rb + 1, pl.ds(aoff + 16 * k, 16)] for k in range(4)]
            vpar2 = vidx_v[pl.ds(rb + L, L)]
            for n in range(2, NV):
                pe = vpar[n] if n < L else vpar2[n - L]
                noff = half_off(pe)
                for k in range(4):
                    acc[k] = acc[k] + vrows[rb + n, pl.ds(noff + 16 * k, 16)]
            pos = u[0] * v[0] + u[1] * v[1] + u[2] * v[2] + u[3] * v[3]
            neg = u[0] * acc[0] + u[1] * acc[1] + u[2] * acc[2] + u[3] * acc[3]
            posb[pl.ds(b * L, L)] = pos
            negb[pl.ds(b * L, L)] = neg
            return carry2

        lax.fori_loop(0, CB, bbody, 0, unroll=False)

        pltpu.sync_copy(posb, pos_hbm.at[pl.ds(gbase * L, CB * L)])
        pltpu.sync_copy(negb, neg_hbm.at[pl.ds(gbase * L, CB * L)])
        return carry

    lax.fori_loop(0, NCHUNK, chunk_body, 0, unroll=False)


_sc_call = functools.partial(
    pl.kernel,
    out_type=(jax.ShapeDtypeStruct((B * L,), jnp.float32),
              jax.ShapeDtypeStruct((B * L,), jnp.float32)),
    mesh=plsc.VectorSubcoreMesh(core_axis_name="c", subcore_axis_name="s"),
    compiler_params=pltpu.CompilerParams(use_tc_tiling_on_sc=True),
    scratch_types=[
        pltpu.VMEM((CB + L,), jnp.int32),
        pltpu.VMEM((CB,), jnp.int32),
        pltpu.VMEM((CB * NV + 2 * L,), jnp.int32),
        pltpu.VMEM((CB * NV,), jnp.int32),
        pltpu.VMEM((CB, W128), jnp.float32),
        pltpu.VMEM((CB * NV, W128), jnp.float32),
        pltpu.VMEM((CB * L,), jnp.float32),
        pltpu.VMEM((CB * L,), jnp.float32),
        pltpu.SemaphoreType.DMA,
    ],
)(_sc_body)


VB = 8192
NGB = 62
TROWS = NGB * VB


def _tr_body(ua_ref, ub_ref, va_ref, vb_ref, uo_ref, vo_ref):
    ii = lax.broadcasted_iota(jnp.int32, (D, W128), 0)
    jj = lax.broadcasted_iota(jnp.int32, (D, W128), 1)
    sel_lo = (ii == jj).astype(jnp.float32)
    sel_hi = (ii == jj - D).astype(jnp.float32)
    dims = (((0,), (0,)), ((), ()))
    for lo, hi, dst in ((ua_ref, ub_ref, uo_ref), (va_ref, vb_ref, vo_ref)):
        dst[...] = (
            lax.dot_general(lo[...], sel_lo, dims,
                            preferred_element_type=jnp.float32)
            + lax.dot_general(hi[...], sel_hi, dims,
                              preferred_element_type=jnp.float32))


_tr_call = pl.pallas_call(
    _tr_body,
    grid=(NGB,),
    in_specs=[pl.BlockSpec((D, VB), lambda j: (0, 2 * j)),
              pl.BlockSpec((D, VB), lambda j: (0, jnp.minimum(2 * j + 1, 122))),
              pl.BlockSpec((D, VB), lambda j: (0, 2 * j)),
              pl.BlockSpec((D, VB), lambda j: (0, jnp.minimum(2 * j + 1, 122)))],
    out_specs=[pl.BlockSpec((VB, W128), lambda j: (j, 0)),
               pl.BlockSpec((VB, W128), lambda j: (j, 0))],
    out_shape=[jax.ShapeDtypeStruct((TROWS, W128), jnp.float32),
               jax.ShapeDtypeStruct((TROWS, W128), jnp.float32)],
)


def _loss_body(pos_ref, neg_ref, out_ref):
    score = jnp.sum(pos_ref[...], axis=1)
    nscore = jnp.sum(neg_ref[...], axis=1)

    def logsig(x):
        return jnp.minimum(x, 0.0) - jnp.log1p(jnp.exp(-jnp.abs(x)))

    out_ref[0, 0] = -jnp.mean(logsig(score) + logsig(-nscore))


_loss_call = pl.pallas_call(
    _loss_body,
    out_shape=jax.ShapeDtypeStruct((1, 1), jnp.float32),
    out_specs=pl.BlockSpec(memory_space=pltpu.SMEM),
)


def kernel(u_pos, v_pos, v_neg, u_weight, v_weight):
    vidx = jnp.concatenate([v_pos[:, None], v_neg], axis=1).reshape(-1)
    uwT, vwT = u_weight.T, v_weight.T
    uw2, vw2 = _tr_call(uwT, uwT, vwT, vwT)
    pos_flat, neg_flat = _sc_call(u_pos, vidx, uw2, vw2)
    out = _loss_call(pos_flat.reshape(B, L), neg_flat.reshape(B, L))
    return out[0, 0]
